# baseline (speedup 1.0000x reference)
import functools

import jax
import jax.numpy as jnp
from jax import lax
from jax.experimental import pallas as pl
from jax.experimental.pallas import tpu_sc as plsc
import jax.experimental.pallas.tpu as pltpu

E = 8
TOP_K = 2
D_MODEL = 2048
H = 2048
NT = 2048
NE = NT * TOP_K
BM = 256
NTILES = NE // BM
TSTEPS = NTILES + E - 1
BH = 256
NHB = H // BH

NC = 2
NS = 16
NW = NC * NS
J1 = NE // NS
J2 = NE // NW
CH = 32
NCHT = J1 // CH
TT = NT // NW


def _router_body(x_ref, wr_ref, logits_ref, ei_ref, gg_ref, meta_ref):
    x = x_ref[...].astype(jnp.bfloat16)
    wr = wr_ref[...].astype(jnp.bfloat16)
    l = lax.dot_general(
        x, wr, (((1,), (1,)), ((), ())),
        preferred_element_type=jnp.float32,
    )
    logits_ref[...] = l
    col = lax.broadcasted_iota(jnp.int32, (NT, E), 1)
    m1 = jnp.max(l, axis=1, keepdims=True)
    i1 = jnp.min(jnp.where(l == m1, col, E), axis=1, keepdims=True)
    lm = jnp.where(col == i1, -jnp.inf, l)
    m2 = jnp.max(lm, axis=1, keepdims=True)
    i2 = jnp.min(jnp.where(lm == m2, col, E), axis=1, keepdims=True)
    e2 = jnp.exp(m2 - m1)
    denom = 1.0 + e2
    ei_ref[...] = jnp.concatenate([i1, i2], axis=1)
    gg_ref[...] = jnp.concatenate([1.0 / denom, e2 / denom], axis=1)

    counts = (jnp.sum((i1 == col).astype(jnp.int32), axis=0, keepdims=True)
              + jnp.sum((i2 == col).astype(jnp.int32), axis=0, keepdims=True))
    lane = lax.broadcasted_iota(jnp.int32, (1, 32), 1)
    zero = jnp.zeros((1, 32), jnp.int32)
    off = zero
    offn = zero
    for e in range(E):
        ce = lax.slice(counts, (0, e), (1, e + 1))
        off = off + jnp.where(lane > e, ce, 0)
        offn = offn + jnp.where(lane >= e, ce, 0)
    sz = offn - off
    st = off // BM
    en = jnp.where(sz > 0, (offn - 1) // BM, -1)
    te = jnp.where(sz > 0, en - st + 1, 0)
    cti = zero
    for e in range(E):
        tee = lax.slice(te, (0, e), (1, e + 1))
        cti = cti + jnp.where(lane >= e, tee, 0)
    se = zero
    for e in range(E):
        ctie = lax.slice(cti, (0, e), (1, e + 1))
        se = se + jnp.where(lane >= ctie, 1, 0)
    se = jnp.minimum(se, E - 1)
    st_of = zero
    ctx_of = zero
    for e in range(E):
        ste = lax.slice(st, (0, e), (1, e + 1))
        ctxe = lax.slice(cti - te, (0, e), (1, e + 1))
        st_of = st_of + jnp.where(se == e, ste, 0)
        ctx_of = ctx_of + jnp.where(se == e, ctxe, 0)
    tile = jnp.clip(st_of + (lane - ctx_of), 0, NTILES - 1)
    tile_prev = jnp.concatenate(
        [jnp.zeros((1, 1), jnp.int32), lax.slice(tile, (0, 0), (1, 31))],
        axis=1)
    fv = jnp.where(lane == 0, 1, jnp.where(tile != tile_prev, 1, 0))
    n_active = lax.slice(cti, (0, E - 1), (1, E))
    fv = jnp.where(lane >= n_active, 2, fv)
    meta_ref[...] = jnp.concatenate([se, tile, fv, off], axis=0)


def _sc_sort_body(ei_hbm, x_hbm, xs_hbm, pos_hbm, cnt_hbm,
                  ei_v, allcnt, pos_flat, pos2, tok2, sem):
    c = lax.axis_index("c")
    s = lax.axis_index("s")
    base1 = s * J1
    pltpu.sync_copy(ei_hbm.at[pl.ds(base1, J1)], ei_v)

    lanes = jnp.arange(16, dtype=jnp.int32)
    cnt = jnp.zeros((16,), jnp.int32)
    for k in range(J1 // 16):
        v = ei_v[pl.ds(k * 16, 16)]
        for e in range(E):
            pc = jnp.sum(jnp.where(v == e, 1, 0).astype(jnp.int32))
            cnt = cnt + jnp.where(lanes == e, pc, 0)
    allcnt[0, :] = cnt
    pltpu.sync_copy(allcnt.at[0], cnt_hbm.at[c, s])
    plsc.subcore_barrier()
    pltpu.sync_copy(cnt_hbm.at[c], allcnt)
    total = jnp.zeros((16,), jnp.int32)
    part = jnp.zeros((16,), jnp.int32)
    for s2 in range(NS):
        c2 = allcnt[s2, :]
        total = total + c2
        before = jnp.full((16,), s2, jnp.int32) < s
        part = part + jnp.where(before, c2, 0)
    off = plsc.cumsum(total) - total
    running = off + part

    for k in range(J1 // 16):
        v = ei_v[pl.ds(k * 16, 16)]
        p_acc = jnp.zeros((16,), jnp.int32)
        for e in range(E):
            m = v == e
            mi = jnp.where(m, 1, 0).astype(jnp.int32)
            rank = plsc.cumsum(mi) - 1
            base_e = jnp.sum(jnp.where(lanes == e, running, 0))
            p_acc = jnp.where(m, base_e + rank, p_acc)
            running = running + jnp.where(lanes == e, jnp.sum(mi), 0)
        pos_flat[pl.ds(k * 16, 16)] = p_acc
        pos2[k // 2, pl.ds((k % 2) * 16, 16)] = p_acc
        tok2[k // 2, pl.ds((k % 2) * 16, 16)] = (base1 + k * 16 + lanes) >> 1

    half = c * (J1 // 2)
    pltpu.sync_copy(pos_flat.at[pl.ds(half, J2)],
                    pos_hbm.at[pl.ds(base1 + half, J2)])

    def chunk(cc, rows_v):
        row = c * (NCHT // 2) + cc
        pltpu.async_copy(x_hbm.at[tok2.at[row]], rows_v, sem).wait()
        pltpu.async_copy(rows_v, xs_hbm.at[pos2.at[row]], sem).wait()

    def body(rows_v):
        for cc in range(NCHT // 2):
            chunk(cc, rows_v)
    pl.run_scoped(body, pltpu.VMEM((CH, D_MODEL), jnp.float32))


def _gmm_body(meta_ref, xs_ref, wi_ref, wo_ref, os_ref):
    s = pl.program_id(0)
    hb = pl.program_id(1)
    e = meta_ref[0, s]
    tile = meta_ref[1, s]
    fv = meta_ref[2, s]
    lo = meta_ref[3, e]
    hi = meta_ref[3, e + 1]
    lo_c = jnp.maximum(lo, tile * BM)
    hi_c = jnp.minimum(hi, tile * BM + BM)

    @pl.when(jnp.logical_and(fv != 2, hi_c > lo_c))
    def _active():
        rowid = tile * BM + lax.broadcasted_iota(jnp.int32, (BM, 1), 0)
        mask = jnp.logical_and(rowid >= lo, rowid < hi)
        xb = xs_ref[...].astype(jnp.bfloat16)
        h = lax.dot_general(
            xb, wi_ref[0], (((1,), (1,)), ((), ())),
            preferred_element_type=jnp.float32,
        ).astype(jnp.bfloat16)
        contrib = lax.dot_general(
            h, wo_ref[0], (((1,), (1,)), ((), ())),
            preferred_element_type=jnp.float32,
        )

        @pl.when(jnp.logical_and(fv == 1, hb == 0))
        def _init():
            os_ref[...] = jnp.where(mask, contrib, 0.0)

        @pl.when(jnp.logical_or(fv == 0, hb != 0))
        def _acc():
            os_ref[...] = jnp.where(mask, os_ref[...] + contrib, os_ref[...])


def _sc_combine_body(os_hbm, pos_hbm, gg_hbm, b_hbm, out_hbm,
                     pos2, gg_v, bias_v, rows_v, out_buf, sem):
    c = lax.axis_index("c")
    s = lax.axis_index("s")
    wid = s * NC + c
    tb = wid * TT
    eb = tb * TOP_K

    def stage(pos_v):
        pltpu.sync_copy(pos_hbm.at[pl.ds(eb, TT * TOP_K)], pos_v)
        for k in range((TT * TOP_K) // 16):
            pos2[k, :] = pos_v[pl.ds(k * 16, 16)]
    pl.run_scoped(stage, pltpu.VMEM((TT * TOP_K,), jnp.int32))
    pltpu.sync_copy(gg_hbm.at[pl.ds(eb, TT * TOP_K)], gg_v)
    pltpu.sync_copy(b_hbm, bias_v)

    lanes = jnp.arange(16, dtype=jnp.int32)
    for cc in range(TT // 16):
        pltpu.async_copy(os_hbm.at[pos2.at[cc * 2]], rows_v.at[0], sem).wait()
        pltpu.async_copy(os_hbm.at[pos2.at[cc * 2 + 1]], rows_v.at[1], sem).wait()

        def tok_body(i, carry):
            ge = gg_v[pl.ds(cc * 32, 16)]
            go = gg_v[pl.ds(cc * 32 + 16, 16)]
            g0 = jnp.sum(jnp.where(lanes == 2 * i, ge, 0.0))
            g1 = jnp.sum(jnp.where(lanes == 2 * i + 1, ge, 0.0))
            h0 = jnp.sum(jnp.where(lanes == 2 * i - 16, go, 0.0))
            h1 = jnp.sum(jnp.where(lanes == 2 * i - 15, go, 0.0))
            use_hi = i >= 8
            g0 = jnp.where(use_hi, h0, g0)
            g1 = jnp.where(use_hi, h1, g1)
            half = jnp.where(use_hi, 1, 0)
            r = 2 * i - 16 * half
            for d in range(D_MODEL // 16):
                sl = pl.ds(d * 16, 16)
                r0 = rows_v[half, r, sl]
                r1 = rows_v[half, r + 1, sl]
                out_buf[i, sl] = g0 * r0 + g1 * r1 + bias_v[sl]
            return carry
        lax.fori_loop(0, 16, tok_body, 0)
        pltpu.sync_copy(out_buf, out_hbm.at[pl.ds(tb + cc * 16, 16)])


@functools.lru_cache(maxsize=1)
def _get_sc_kernels():
    mesh = plsc.VectorSubcoreMesh(
        core_axis_name="c", subcore_axis_name="s",
        num_cores=NC, num_subcores=NS)
    params = pltpu.CompilerParams(needs_layout_passes=False)
    sort_k = pl.kernel(
        _sc_sort_body,
        out_type=(
            jax.ShapeDtypeStruct((NE, D_MODEL), jnp.float32),
            jax.ShapeDtypeStruct((NE,), jnp.int32),
            jax.ShapeDtypeStruct((NC, NS, 16), jnp.int32),
        ),
        mesh=mesh,
        compiler_params=params,
        scratch_types=[
            pltpu.VMEM((J1,), jnp.int32),
            pltpu.VMEM((NS, 16), jnp.int32),
            pltpu.VMEM((J1,), jnp.int32),
            pltpu.VMEM((NCHT, CH), jnp.int32),
            pltpu.VMEM((NCHT, CH), jnp.int32),
            pltpu.SemaphoreType.DMA,
        ],
    )
    comb_k = pl.kernel(
        _sc_combine_body,
        out_type=jax.ShapeDtypeStruct((NT, D_MODEL), jnp.float32),
        mesh=mesh,
        compiler_params=params,
        scratch_types=[
            pltpu.VMEM((TT * TOP_K // 16, 16), jnp.int32),
            pltpu.VMEM((TT * TOP_K,), jnp.float32),
            pltpu.VMEM((D_MODEL,), jnp.float32),
            pltpu.VMEM((2, 16, D_MODEL), jnp.float32),
            pltpu.VMEM((16, D_MODEL), jnp.float32),
            pltpu.SemaphoreType.DMA,
        ],
    )
    return sort_k, comb_k


@jax.jit
def kernel(layer_input, W_in, W_out, Wr, bias):
    x = layer_input.reshape(NT, D_MODEL)

    logits, ei, gg, meta = pl.pallas_call(
        _router_body,
        out_shape=(
            jax.ShapeDtypeStruct((NT, E), jnp.float32),
            jax.ShapeDtypeStruct((NT, TOP_K), jnp.int32),
            jax.ShapeDtypeStruct((NT, TOP_K), jnp.float32),
            jax.ShapeDtypeStruct((4, 32), jnp.int32),
        ),
    )(x, Wr)

    sort_k, comb_k = _get_sc_kernels()
    xs, pos, _cnt = sort_k(ei.reshape(NE), x)

    wi = W_in.astype(jnp.bfloat16)
    wo = W_out.astype(jnp.bfloat16)

    os_sorted = pl.pallas_call(
        _gmm_body,
        grid_spec=pltpu.PrefetchScalarGridSpec(
            num_scalar_prefetch=1,
            grid=(TSTEPS, NHB),
            in_specs=[
                pl.BlockSpec((BM, D_MODEL), lambda s, hb, m: (m[1, s], 0)),
                pl.BlockSpec((1, BH, D_MODEL), lambda s, hb, m: (m[0, s], hb, 0)),
                pl.BlockSpec((1, D_MODEL, BH), lambda s, hb, m: (m[0, s], 0, hb)),
            ],
            out_specs=pl.BlockSpec((BM, D_MODEL), lambda s, hb, m: (m[1, s], 0)),
        ),
        out_shape=jax.ShapeDtypeStruct((NE, D_MODEL), jnp.float32),
    )(meta, xs, wi, wo)

    out = comb_k(os_sorted, pos, gg.reshape(NE), bias)

    return (out.reshape(1, NT, D_MODEL), logits)


# speedup vs baseline: 1.4716x; 1.4716x over previous
import functools

import jax
import jax.numpy as jnp
from jax import lax
from jax.experimental import pallas as pl
from jax.experimental.pallas import tpu_sc as plsc
import jax.experimental.pallas.tpu as pltpu

E = 8
TOP_K = 2
D_MODEL = 2048
H = 2048
NT = 2048
NE = NT * TOP_K
BM = 256
NTILES = NE // BM
TSTEPS = NTILES + E - 1
BH = 256
NHB = H // BH

NC = 2
NS = 16
NW = NC * NS
J1 = NE // NS
J2 = NE // NW
CH = 32
NCHT = J1 // CH
TT = NT // NW


def _router_body(x_ref, wr_ref, logits_ref, ei_ref, gg_ref, meta_ref):
    x = x_ref[...].astype(jnp.bfloat16)
    wr = wr_ref[...].astype(jnp.bfloat16)
    l = lax.dot_general(
        x, wr, (((1,), (1,)), ((), ())),
        preferred_element_type=jnp.float32,
    )
    logits_ref[...] = l
    col = lax.broadcasted_iota(jnp.int32, (NT, E), 1)
    m1 = jnp.max(l, axis=1, keepdims=True)
    i1 = jnp.min(jnp.where(l == m1, col, E), axis=1, keepdims=True)
    lm = jnp.where(col == i1, -jnp.inf, l)
    m2 = jnp.max(lm, axis=1, keepdims=True)
    i2 = jnp.min(jnp.where(lm == m2, col, E), axis=1, keepdims=True)
    e2 = jnp.exp(m2 - m1)
    denom = 1.0 + e2
    ei_ref[...] = jnp.concatenate([i1, i2], axis=1)
    gg_ref[...] = jnp.concatenate([1.0 / denom, e2 / denom], axis=1)

    counts = (jnp.sum((i1 == col).astype(jnp.int32), axis=0, keepdims=True)
              + jnp.sum((i2 == col).astype(jnp.int32), axis=0, keepdims=True))
    lane = lax.broadcasted_iota(jnp.int32, (1, 32), 1)
    zero = jnp.zeros((1, 32), jnp.int32)
    off = zero
    offn = zero
    for e in range(E):
        ce = lax.slice(counts, (0, e), (1, e + 1))
        off = off + jnp.where(lane > e, ce, 0)
        offn = offn + jnp.where(lane >= e, ce, 0)
    sz = offn - off
    st = off // BM
    en = jnp.where(sz > 0, (offn - 1) // BM, -1)
    te = jnp.where(sz > 0, en - st + 1, 0)
    cti = zero
    for e in range(E):
        tee = lax.slice(te, (0, e), (1, e + 1))
        cti = cti + jnp.where(lane >= e, tee, 0)
    se = zero
    for e in range(E):
        ctie = lax.slice(cti, (0, e), (1, e + 1))
        se = se + jnp.where(lane >= ctie, 1, 0)
    se = jnp.minimum(se, E - 1)
    st_of = zero
    ctx_of = zero
    for e in range(E):
        ste = lax.slice(st, (0, e), (1, e + 1))
        ctxe = lax.slice(cti - te, (0, e), (1, e + 1))
        st_of = st_of + jnp.where(se == e, ste, 0)
        ctx_of = ctx_of + jnp.where(se == e, ctxe, 0)
    tile = jnp.clip(st_of + (lane - ctx_of), 0, NTILES - 1)
    tile_prev = jnp.concatenate(
        [jnp.zeros((1, 1), jnp.int32), lax.slice(tile, (0, 0), (1, 31))],
        axis=1)
    fv = jnp.where(lane == 0, 1, jnp.where(tile != tile_prev, 1, 0))
    n_active = lax.slice(cti, (0, E - 1), (1, E))
    fv = jnp.where(lane >= n_active, 2, fv)
    meta_ref[...] = jnp.concatenate([se, tile, fv, off], axis=0)


def _sc_sort_body(ei_hbm, x_hbm, xs_hbm, pos_hbm, cnt_hbm,
                  ei_v, allcnt, pos_flat, pos2, tok2, sem):
    c = lax.axis_index("c")
    s = lax.axis_index("s")
    base1 = s * J1
    pltpu.sync_copy(ei_hbm.at[pl.ds(base1, J1)], ei_v)

    lanes = jnp.arange(16, dtype=jnp.int32)
    cnt = jnp.zeros((16,), jnp.int32)
    for k in range(J1 // 16):
        v = ei_v[pl.ds(k * 16, 16)]
        for e in range(E):
            pc = jnp.sum(jnp.where(v == e, 1, 0).astype(jnp.int32))
            cnt = cnt + jnp.where(lanes == e, pc, 0)
    allcnt[0, :] = cnt
    pltpu.sync_copy(allcnt.at[0], cnt_hbm.at[c, s])
    plsc.subcore_barrier()
    pltpu.sync_copy(cnt_hbm.at[c], allcnt)
    total = jnp.zeros((16,), jnp.int32)
    part = jnp.zeros((16,), jnp.int32)
    for s2 in range(NS):
        c2 = allcnt[s2, :]
        total = total + c2
        before = jnp.full((16,), s2, jnp.int32) < s
        part = part + jnp.where(before, c2, 0)
    off = plsc.cumsum(total) - total
    running = off + part

    for k in range(J1 // 16):
        v = ei_v[pl.ds(k * 16, 16)]
        p_acc = jnp.zeros((16,), jnp.int32)
        for e in range(E):
            m = v == e
            mi = jnp.where(m, 1, 0).astype(jnp.int32)
            rank = plsc.cumsum(mi) - 1
            base_e = jnp.sum(jnp.where(lanes == e, running, 0))
            p_acc = jnp.where(m, base_e + rank, p_acc)
            running = running + jnp.where(lanes == e, jnp.sum(mi), 0)
        pos_flat[pl.ds(k * 16, 16)] = p_acc
        pos2[k // 2, pl.ds((k % 2) * 16, 16)] = p_acc
        tok2[k // 2, pl.ds((k % 2) * 16, 16)] = (base1 + k * 16 + lanes) >> 1

    half = c * (J1 // 2)
    pltpu.sync_copy(pos_flat.at[pl.ds(half, J2)],
                    pos_hbm.at[pl.ds(base1 + half, J2)])

    def chunk(cc, rows_v):
        row = c * (NCHT // 2) + cc
        pltpu.async_copy(x_hbm.at[tok2.at[row]], rows_v, sem).wait()
        pltpu.async_copy(rows_v, xs_hbm.at[pos2.at[row]], sem).wait()

    def body(rows_v):
        for cc in range(NCHT // 2):
            chunk(cc, rows_v)
    pl.run_scoped(body, pltpu.VMEM((CH, D_MODEL), jnp.float32))


def _gmm_body(meta_ref, xs_ref, wi_ref, wo_ref, os_ref):
    s = pl.program_id(0)
    e = meta_ref[0, s]
    tile = meta_ref[1, s]
    fv = meta_ref[2, s]
    lo = meta_ref[3, e]
    hi = meta_ref[3, e + 1]
    lo_c = jnp.maximum(lo, tile * BM)
    hi_c = jnp.minimum(hi, tile * BM + BM)

    @pl.when(jnp.logical_and(fv != 2, hi_c > lo_c))
    def _active():
        rowid = tile * BM + lax.broadcasted_iota(jnp.int32, (BM, 1), 0)
        mask = jnp.logical_and(rowid >= lo, rowid < hi)
        xb = xs_ref[...].astype(jnp.bfloat16)
        h = lax.dot_general(
            xb, wi_ref[0], (((1,), (1,)), ((), ())),
            preferred_element_type=jnp.float32,
        ).astype(jnp.bfloat16)
        contrib = lax.dot_general(
            h, wo_ref[0], (((1,), (1,)), ((), ())),
            preferred_element_type=jnp.float32,
        )

        @pl.when(fv == 1)
        def _init():
            os_ref[...] = jnp.where(mask, contrib, 0.0)

        @pl.when(fv == 0)
        def _acc():
            os_ref[...] = jnp.where(mask, contrib, os_ref[...])


def _sc_combine_body(os_hbm, pos_hbm, gg_hbm, b_hbm, out_hbm,
                     pos2, gg_v, bias_v, rows_v, out_buf, sem):
    c = lax.axis_index("c")
    s = lax.axis_index("s")
    wid = s * NC + c
    tb = wid * TT
    eb = tb * TOP_K

    def stage(pos_v):
        pltpu.sync_copy(pos_hbm.at[pl.ds(eb, TT * TOP_K)], pos_v)
        for k in range((TT * TOP_K) // 16):
            pos2[k, :] = pos_v[pl.ds(k * 16, 16)]
    pl.run_scoped(stage, pltpu.VMEM((TT * TOP_K,), jnp.int32))
    pltpu.sync_copy(gg_hbm.at[pl.ds(eb, TT * TOP_K)], gg_v)
    pltpu.sync_copy(b_hbm, bias_v)

    lanes = jnp.arange(16, dtype=jnp.int32)
    for cc in range(TT // 16):
        pltpu.async_copy(os_hbm.at[pos2.at[cc * 2]], rows_v.at[0], sem).wait()
        pltpu.async_copy(os_hbm.at[pos2.at[cc * 2 + 1]], rows_v.at[1], sem).wait()

        def tok_body(i, carry):
            ge = gg_v[pl.ds(cc * 32, 16)]
            go = gg_v[pl.ds(cc * 32 + 16, 16)]
            g0 = jnp.sum(jnp.where(lanes == 2 * i, ge, 0.0))
            g1 = jnp.sum(jnp.where(lanes == 2 * i + 1, ge, 0.0))
            h0 = jnp.sum(jnp.where(lanes == 2 * i - 16, go, 0.0))
            h1 = jnp.sum(jnp.where(lanes == 2 * i - 15, go, 0.0))
            use_hi = i >= 8
            g0 = jnp.where(use_hi, h0, g0)
            g1 = jnp.where(use_hi, h1, g1)
            half = jnp.where(use_hi, 1, 0)
            r = 2 * i - 16 * half
            for d in range(D_MODEL // 16):
                sl = pl.ds(d * 16, 16)
                r0 = rows_v[half, r, sl]
                r1 = rows_v[half, r + 1, sl]
                out_buf[i, sl] = g0 * r0 + g1 * r1 + bias_v[sl]
            return carry
        lax.fori_loop(0, 16, tok_body, 0)
        pltpu.sync_copy(out_buf, out_hbm.at[pl.ds(tb + cc * 16, 16)])


@functools.lru_cache(maxsize=1)
def _get_sc_kernels():
    mesh = plsc.VectorSubcoreMesh(
        core_axis_name="c", subcore_axis_name="s",
        num_cores=NC, num_subcores=NS)
    params = pltpu.CompilerParams(needs_layout_passes=False)
    sort_k = pl.kernel(
        _sc_sort_body,
        out_type=(
            jax.ShapeDtypeStruct((NE, D_MODEL), jnp.float32),
            jax.ShapeDtypeStruct((NE,), jnp.int32),
            jax.ShapeDtypeStruct((NC, NS, 16), jnp.int32),
        ),
        mesh=mesh,
        compiler_params=params,
        scratch_types=[
            pltpu.VMEM((J1,), jnp.int32),
            pltpu.VMEM((NS, 16), jnp.int32),
            pltpu.VMEM((J1,), jnp.int32),
            pltpu.VMEM((NCHT, CH), jnp.int32),
            pltpu.VMEM((NCHT, CH), jnp.int32),
            pltpu.SemaphoreType.DMA,
        ],
    )
    comb_k = pl.kernel(
        _sc_combine_body,
        out_type=jax.ShapeDtypeStruct((NT, D_MODEL), jnp.float32),
        mesh=mesh,
        compiler_params=params,
        scratch_types=[
            pltpu.VMEM((TT * TOP_K // 16, 16), jnp.int32),
            pltpu.VMEM((TT * TOP_K,), jnp.float32),
            pltpu.VMEM((D_MODEL,), jnp.float32),
            pltpu.VMEM((2, 16, D_MODEL), jnp.float32),
            pltpu.VMEM((16, D_MODEL), jnp.float32),
            pltpu.SemaphoreType.DMA,
        ],
    )
    return sort_k, comb_k


@jax.jit
def kernel(layer_input, W_in, W_out, Wr, bias):
    x = layer_input.reshape(NT, D_MODEL)

    logits, ei, gg, meta = pl.pallas_call(
        _router_body,
        out_shape=(
            jax.ShapeDtypeStruct((NT, E), jnp.float32),
            jax.ShapeDtypeStruct((NT, TOP_K), jnp.int32),
            jax.ShapeDtypeStruct((NT, TOP_K), jnp.float32),
            jax.ShapeDtypeStruct((4, 32), jnp.int32),
        ),
    )(x, Wr)

    sort_k, comb_k = _get_sc_kernels()
    xs, pos, _cnt = sort_k(ei.reshape(NE), x)

    wi = W_in.astype(jnp.bfloat16)
    wo = W_out.astype(jnp.bfloat16)

    os_sorted = pl.pallas_call(
        _gmm_body,
        grid_spec=pltpu.PrefetchScalarGridSpec(
            num_scalar_prefetch=1,
            grid=(TSTEPS,),
            in_specs=[
                pl.BlockSpec((BM, D_MODEL), lambda s, m: (m[1, s], 0)),
                pl.BlockSpec((1, H, D_MODEL), lambda s, m: (m[0, s], 0, 0)),
                pl.BlockSpec((1, D_MODEL, H), lambda s, m: (m[0, s], 0, 0)),
            ],
            out_specs=pl.BlockSpec((BM, D_MODEL), lambda s, m: (m[1, s], 0)),
        ),
        out_shape=jax.ShapeDtypeStruct((NE, D_MODEL), jnp.float32),
    )(meta, xs, wi, wo)

    out = comb_k(os_sorted, pos, gg.reshape(NE), bias)

    return (out.reshape(1, NT, D_MODEL), logits)


# speedup vs baseline: 1.4771x; 1.0038x over previous
import functools

import jax
import jax.numpy as jnp
from jax import lax
from jax.experimental import pallas as pl
from jax.experimental.pallas import tpu_sc as plsc
import jax.experimental.pallas.tpu as pltpu

E = 8
TOP_K = 2
D_MODEL = 2048
H = 2048
NT = 2048
NE = NT * TOP_K
BM = 256
NTILES = NE // BM
TSTEPS = NTILES + E - 1
BH = 256
NHB = H // BH

NC = 2
NS = 16
NW = NC * NS
J1 = NE // NS
J2 = NE // NW
CH = 32
NCHT = J1 // CH
TT = NT // NW


def _router_body(x_ref, wr_ref, logits_ref, ei_ref, gg_ref, meta_ref):
    x = x_ref[...].astype(jnp.bfloat16)
    wr = wr_ref[...].astype(jnp.bfloat16)
    l = lax.dot_general(
        x, wr, (((1,), (1,)), ((), ())),
        preferred_element_type=jnp.float32,
    )
    logits_ref[...] = l
    col = lax.broadcasted_iota(jnp.int32, (NT, E), 1)
    m1 = jnp.max(l, axis=1, keepdims=True)
    i1 = jnp.min(jnp.where(l == m1, col, E), axis=1, keepdims=True)
    lm = jnp.where(col == i1, -jnp.inf, l)
    m2 = jnp.max(lm, axis=1, keepdims=True)
    i2 = jnp.min(jnp.where(lm == m2, col, E), axis=1, keepdims=True)
    e2 = jnp.exp(m2 - m1)
    denom = 1.0 + e2
    ei_ref[...] = jnp.concatenate([i1, i2], axis=1)
    gg_ref[...] = jnp.concatenate([1.0 / denom, e2 / denom], axis=1)

    counts = (jnp.sum((i1 == col).astype(jnp.int32), axis=0, keepdims=True)
              + jnp.sum((i2 == col).astype(jnp.int32), axis=0, keepdims=True))
    lane = lax.broadcasted_iota(jnp.int32, (1, 32), 1)
    zero = jnp.zeros((1, 32), jnp.int32)
    off = zero
    offn = zero
    for e in range(E):
        ce = lax.slice(counts, (0, e), (1, e + 1))
        off = off + jnp.where(lane > e, ce, 0)
        offn = offn + jnp.where(lane >= e, ce, 0)
    sz = offn - off
    st = off // BM
    en = jnp.where(sz > 0, (offn - 1) // BM, -1)
    te = jnp.where(sz > 0, en - st + 1, 0)
    cti = zero
    for e in range(E):
        tee = lax.slice(te, (0, e), (1, e + 1))
        cti = cti + jnp.where(lane >= e, tee, 0)
    se = zero
    for e in range(E):
        ctie = lax.slice(cti, (0, e), (1, e + 1))
        se = se + jnp.where(lane >= ctie, 1, 0)
    se = jnp.minimum(se, E - 1)
    st_of = zero
    ctx_of = zero
    for e in range(E):
        ste = lax.slice(st, (0, e), (1, e + 1))
        ctxe = lax.slice(cti - te, (0, e), (1, e + 1))
        st_of = st_of + jnp.where(se == e, ste, 0)
        ctx_of = ctx_of + jnp.where(se == e, ctxe, 0)
    tile = jnp.clip(st_of + (lane - ctx_of), 0, NTILES - 1)
    tile_prev = jnp.concatenate(
        [jnp.zeros((1, 1), jnp.int32), lax.slice(tile, (0, 0), (1, 31))],
        axis=1)
    fv = jnp.where(lane == 0, 1, jnp.where(tile != tile_prev, 1, 0))
    n_active = lax.slice(cti, (0, E - 1), (1, E))
    fv = jnp.where(lane >= n_active, 2, fv)
    meta_ref[...] = jnp.concatenate([se, tile, fv, off], axis=0)


def _sc_sort_body(ei_hbm, x_hbm, xs_hbm, pos_hbm, cnt_hbm,
                  ei_v, allcnt, pos_flat, pos2, tok2):
    c = lax.axis_index("c")
    s = lax.axis_index("s")
    base1 = s * J1
    pltpu.sync_copy(ei_hbm.at[pl.ds(base1, J1)], ei_v)

    lanes = jnp.arange(16, dtype=jnp.int32)
    cnt = jnp.zeros((16,), jnp.int32)
    for k in range(J1 // 16):
        v = ei_v[pl.ds(k * 16, 16)]
        for e in range(E):
            pc = jnp.sum(jnp.where(v == e, 1, 0).astype(jnp.int32))
            cnt = cnt + jnp.where(lanes == e, pc, 0)
    allcnt[0, :] = cnt
    pltpu.sync_copy(allcnt.at[0], cnt_hbm.at[c, s])
    plsc.subcore_barrier()
    pltpu.sync_copy(cnt_hbm.at[c], allcnt)
    total = jnp.zeros((16,), jnp.int32)
    part = jnp.zeros((16,), jnp.int32)
    for s2 in range(NS):
        c2 = allcnt[s2, :]
        total = total + c2
        before = jnp.full((16,), s2, jnp.int32) < s
        part = part + jnp.where(before, c2, 0)
    off = plsc.cumsum(total) - total
    running = off + part

    for k in range(J1 // 16):
        v = ei_v[pl.ds(k * 16, 16)]
        p_acc = jnp.zeros((16,), jnp.int32)
        for e in range(E):
            m = v == e
            mi = jnp.where(m, 1, 0).astype(jnp.int32)
            rank = plsc.cumsum(mi) - 1
            base_e = jnp.sum(jnp.where(lanes == e, running, 0))
            p_acc = jnp.where(m, base_e + rank, p_acc)
            running = running + jnp.where(lanes == e, jnp.sum(mi), 0)
        pos_flat[pl.ds(k * 16, 16)] = p_acc
        pos2[k, :] = p_acc
        tok2[k, :] = (base1 + k * 16 + lanes) >> 1

    half = c * (J1 // 2)
    pltpu.sync_copy(pos_flat.at[pl.ds(half, J2)],
                    pos_hbm.at[pl.ds(base1 + half, J2)])

    NCH2 = J2 // 16

    def body(rows2, sg0, sg1, ss0, ss1):
        sg = (sg0, sg1)
        ss = (ss0, ss1)

        def g_desc(cc):
            row = c * NCH2 + cc
            return pltpu.async_copy(
                x_hbm.at[tok2.at[row]], rows2.at[cc % 2], sg[cc % 2])

        def s_desc(cc):
            row = c * NCH2 + cc
            return pltpu.async_copy(
                rows2.at[cc % 2], xs_hbm.at[pos2.at[row]], ss[cc % 2])

        scats = [None] * NCH2
        gd = g_desc(0)
        for cc in range(NCH2):
            gd.wait()
            scats[cc] = s_desc(cc)
            if cc + 1 < NCH2:
                if cc >= 1:
                    scats[cc - 1].wait()
                gd = g_desc(cc + 1)
        scats[NCH2 - 2].wait()
        scats[NCH2 - 1].wait()

    pl.run_scoped(body, pltpu.VMEM((2, 16, D_MODEL), jnp.float32),
                  pltpu.SemaphoreType.DMA, pltpu.SemaphoreType.DMA,
                  pltpu.SemaphoreType.DMA, pltpu.SemaphoreType.DMA)


def _gmm_body(meta_ref, xs_ref, wi_ref, wo_ref, os_ref):
    s = pl.program_id(0)
    e = meta_ref[0, s]
    tile = meta_ref[1, s]
    fv = meta_ref[2, s]
    lo = meta_ref[3, e]
    hi = meta_ref[3, e + 1]
    lo_c = jnp.maximum(lo, tile * BM)
    hi_c = jnp.minimum(hi, tile * BM + BM)

    @pl.when(jnp.logical_and(fv != 2, hi_c > lo_c))
    def _active():
        rowid = tile * BM + lax.broadcasted_iota(jnp.int32, (BM, 1), 0)
        mask = jnp.logical_and(rowid >= lo, rowid < hi)
        xb = xs_ref[...].astype(jnp.bfloat16)
        h = lax.dot_general(
            xb, wi_ref[0], (((1,), (1,)), ((), ())),
            preferred_element_type=jnp.float32,
        ).astype(jnp.bfloat16)
        contrib = lax.dot_general(
            h, wo_ref[0], (((1,), (1,)), ((), ())),
            preferred_element_type=jnp.float32,
        )

        @pl.when(fv == 1)
        def _init():
            os_ref[...] = jnp.where(mask, contrib, 0.0)

        @pl.when(fv == 0)
        def _acc():
            os_ref[...] = jnp.where(mask, contrib, os_ref[...])


def _sc_combine_body(os_hbm, pos_hbm, gg_hbm, b_hbm, out_hbm,
                     pos2, gg_v, bias_v, rows_v, out_buf, sem):
    c = lax.axis_index("c")
    s = lax.axis_index("s")
    wid = s * NC + c
    tb = wid * TT
    eb = tb * TOP_K

    def stage(pos_v):
        pltpu.sync_copy(pos_hbm.at[pl.ds(eb, TT * TOP_K)], pos_v)
        for k in range((TT * TOP_K) // 16):
            pos2[k, :] = pos_v[pl.ds(k * 16, 16)]
    pl.run_scoped(stage, pltpu.VMEM((TT * TOP_K,), jnp.int32))
    pltpu.sync_copy(gg_hbm.at[pl.ds(eb, TT * TOP_K)], gg_v)
    pltpu.sync_copy(b_hbm, bias_v)

    lanes = jnp.arange(16, dtype=jnp.int32)
    for cc in range(TT // 16):
        pltpu.async_copy(os_hbm.at[pos2.at[cc * 2]], rows_v.at[0], sem).wait()
        pltpu.async_copy(os_hbm.at[pos2.at[cc * 2 + 1]], rows_v.at[1], sem).wait()

        def tok_body(i, carry):
            ge = gg_v[pl.ds(cc * 32, 16)]
            go = gg_v[pl.ds(cc * 32 + 16, 16)]
            g0 = jnp.sum(jnp.where(lanes == 2 * i, ge, 0.0))
            g1 = jnp.sum(jnp.where(lanes == 2 * i + 1, ge, 0.0))
            h0 = jnp.sum(jnp.where(lanes == 2 * i - 16, go, 0.0))
            h1 = jnp.sum(jnp.where(lanes == 2 * i - 15, go, 0.0))
            use_hi = i >= 8
            g0 = jnp.where(use_hi, h0, g0)
            g1 = jnp.where(use_hi, h1, g1)
            half = jnp.where(use_hi, 1, 0)
            r = 2 * i - 16 * half
            for d in range(D_MODEL // 16):
                sl = pl.ds(d * 16, 16)
                r0 = rows_v[half, r, sl]
                r1 = rows_v[half, r + 1, sl]
                out_buf[i, sl] = g0 * r0 + g1 * r1 + bias_v[sl]
            return carry
        lax.fori_loop(0, 16, tok_body, 0)
        pltpu.sync_copy(out_buf, out_hbm.at[pl.ds(tb + cc * 16, 16)])


@functools.lru_cache(maxsize=1)
def _get_sc_kernels():
    mesh = plsc.VectorSubcoreMesh(
        core_axis_name="c", subcore_axis_name="s",
        num_cores=NC, num_subcores=NS)
    params = pltpu.CompilerParams(needs_layout_passes=False)
    sort_k = pl.kernel(
        _sc_sort_body,
        out_type=(
            jax.ShapeDtypeStruct((NE, D_MODEL), jnp.float32),
            jax.ShapeDtypeStruct((NE,), jnp.int32),
            jax.ShapeDtypeStruct((NC, NS, 16), jnp.int32),
        ),
        mesh=mesh,
        compiler_params=params,
        scratch_types=[
            pltpu.VMEM((J1,), jnp.int32),
            pltpu.VMEM((NS, 16), jnp.int32),
            pltpu.VMEM((J1,), jnp.int32),
            pltpu.VMEM((J1 // 16, 16), jnp.int32),
            pltpu.VMEM((J1 // 16, 16), jnp.int32),
        ],
    )
    comb_k = pl.kernel(
        _sc_combine_body,
        out_type=jax.ShapeDtypeStruct((NT, D_MODEL), jnp.float32),
        mesh=mesh,
        compiler_params=params,
        scratch_types=[
            pltpu.VMEM((TT * TOP_K // 16, 16), jnp.int32),
            pltpu.VMEM((TT * TOP_K,), jnp.float32),
            pltpu.VMEM((D_MODEL,), jnp.float32),
            pltpu.VMEM((2, 16, D_MODEL), jnp.float32),
            pltpu.VMEM((16, D_MODEL), jnp.float32),
            pltpu.SemaphoreType.DMA,
        ],
    )
    return sort_k, comb_k


@jax.jit
def kernel(layer_input, W_in, W_out, Wr, bias):
    x = layer_input.reshape(NT, D_MODEL)

    logits, ei, gg, meta = pl.pallas_call(
        _router_body,
        out_shape=(
            jax.ShapeDtypeStruct((NT, E), jnp.float32),
            jax.ShapeDtypeStruct((NT, TOP_K), jnp.int32),
            jax.ShapeDtypeStruct((NT, TOP_K), jnp.float32),
            jax.ShapeDtypeStruct((4, 32), jnp.int32),
        ),
    )(x, Wr)

    sort_k, comb_k = _get_sc_kernels()
    xs, pos, _cnt = sort_k(ei.reshape(NE), x)

    wi = W_in.astype(jnp.bfloat16)
    wo = W_out.astype(jnp.bfloat16)

    os_sorted = pl.pallas_call(
        _gmm_body,
        grid_spec=pltpu.PrefetchScalarGridSpec(
            num_scalar_prefetch=1,
            grid=(TSTEPS,),
            in_specs=[
                pl.BlockSpec((BM, D_MODEL), lambda s, m: (m[1, s], 0)),
                pl.BlockSpec((1, H, D_MODEL), lambda s, m: (m[0, s], 0, 0)),
                pl.BlockSpec((1, D_MODEL, H), lambda s, m: (m[0, s], 0, 0)),
            ],
            out_specs=pl.BlockSpec((BM, D_MODEL), lambda s, m: (m[1, s], 0)),
        ),
        out_shape=jax.ShapeDtypeStruct((NE, D_MODEL), jnp.float32),
    )(meta, xs, wi, wo)

    out = comb_k(os_sorted, pos, gg.reshape(NE), bias)

    return (out.reshape(1, NT, D_MODEL), logits)
